# baseline (device time: 306613 ns/iter reference)
import jax
import jax.numpy as jnp
from jax import lax
from jax.experimental import pallas as pl
from jax.experimental.pallas import tpu as pltpu

N_DEV = 4
N_TOK = 4096
D_MODEL = 1024
H = 2048
HB = H // 2
E_LOCAL = 4
C = 1280
N_HOPS = N_DEV - 1


def kernel(x, router_W, route_idx, expert_W):
    del router_W
    my = lax.axis_index("i")

    ridx = route_idx[:, 0]
    order = jnp.argsort(ridx, stable=True)
    grp = ridx[order] // E_LOCAL
    gstart = jnp.searchsorted(grp, jnp.arange(N_DEV))
    order_pad = jnp.concatenate([order, jnp.zeros((C,), jnp.int32)])
    tok = lax.dynamic_slice(order_pad, (gstart[my],), (C,))
    n_mine = jnp.searchsorted(grp, my + 1) - gstart[my]
    valid = jnp.arange(C) < n_mine
    lidx = jnp.where(valid, ridx[tok] - my * E_LOCAL, -1).astype(jnp.int32)

    xg = jnp.take(x, tok, axis=0).astype(jnp.bfloat16)
    wb = expert_W.astype(jnp.bfloat16).reshape(E_LOCAL * D_MODEL, H)

    def body(xg_ref, lidx_ref, w_ref, out_ref, xcat_ref, comm_cw, comm_ccw,
             send_cw, recv_cw, send_ccw, recv_ccw, copy_sems):
        me = lax.axis_index("i")
        left = lax.rem(me + N_DEV - 1, N_DEV)
        right = lax.rem(me + 1, N_DEV)

        barrier_sem = pltpu.get_barrier_semaphore()
        for nbr in (left, right):
            pl.semaphore_signal(
                barrier_sem, inc=1,
                device_id=(nbr,), device_id_type=pl.DeviceIdType.MESH,
            )
        pl.semaphore_wait(barrier_sem, 2)

        for le in range(E_LOCAL):
            xcat_ref[:, pl.ds(le * D_MODEL, D_MODEL)] = jnp.where(
                lidx_ref[...] == le, xg_ref[...], jnp.zeros_like(xg_ref)
            )
        for half, slot0 in ((0, comm_cw.at[0]), (1, comm_ccw.at[0])):
            slot0[...] = jnp.dot(
                xcat_ref[...],
                w_ref[:, pl.ds(half * HB, HB)],
                preferred_element_type=jnp.float32,
            ).astype(jnp.bfloat16)

        def store_half(origin, src, half, sem_idx):
            cp = pltpu.make_async_copy(
                src,
                out_ref.at[origin, :, pl.ds(half * HB, HB)],
                copy_sems.at[sem_idx],
            )
            cp.start()
            return cp

        cp0 = store_half(me, comm_cw.at[0], 0, 0)
        cp1 = store_half(me, comm_ccw.at[0], 1, 1)

        for t in range(N_HOPS):
            send_slot = t % 2
            recv_slot = (t + 1) % 2
            rdma_cw = pltpu.make_async_remote_copy(
                src_ref=comm_cw.at[send_slot],
                dst_ref=comm_cw.at[recv_slot],
                send_sem=send_cw.at[t],
                recv_sem=recv_cw.at[t],
                device_id=(right,),
                device_id_type=pl.DeviceIdType.MESH,
            )
            rdma_ccw = pltpu.make_async_remote_copy(
                src_ref=comm_ccw.at[send_slot],
                dst_ref=comm_ccw.at[recv_slot],
                send_sem=send_ccw.at[t],
                recv_sem=recv_ccw.at[t],
                device_id=(left,),
                device_id_type=pl.DeviceIdType.MESH,
            )
            rdma_cw.start()
            rdma_ccw.start()
            if t == 0:
                cp0.wait()
                cp1.wait()
            rdma_cw.wait()
            rdma_ccw.wait()
            o_cw = lax.rem(me - 1 - t + 2 * N_DEV, N_DEV)
            o_ccw = lax.rem(me + 1 + t, N_DEV)
            cp0 = store_half(o_cw, comm_cw.at[recv_slot], 0, 0)
            cp1 = store_half(o_ccw, comm_ccw.at[recv_slot], 1, 1)
            cp0.wait()
            cp1.wait()

    blocks = pl.pallas_call(
        body,
        out_shape=jax.ShapeDtypeStruct((N_DEV, C, H), jnp.bfloat16),
        in_specs=[
            pl.BlockSpec(memory_space=pltpu.VMEM),
            pl.BlockSpec(memory_space=pltpu.VMEM),
            pl.BlockSpec(memory_space=pltpu.VMEM),
        ],
        out_specs=pl.BlockSpec(memory_space=pl.ANY),
        scratch_shapes=[
            pltpu.VMEM((C, E_LOCAL * D_MODEL), jnp.bfloat16),
            pltpu.VMEM((2, C, HB), jnp.bfloat16),
            pltpu.VMEM((2, C, HB), jnp.bfloat16),
            pltpu.SemaphoreType.DMA((N_HOPS,)),
            pltpu.SemaphoreType.DMA((N_HOPS,)),
            pltpu.SemaphoreType.DMA((N_HOPS,)),
            pltpu.SemaphoreType.DMA((N_HOPS,)),
            pltpu.SemaphoreType.DMA((2,)),
        ],
        compiler_params=pltpu.CompilerParams(
            collective_id=0, vmem_limit_bytes=100 * 1024 * 1024
        ),
    )(xg, lidx[:, None], wb)

    inv = jnp.zeros((N_TOK,), jnp.int32).at[order].set(
        jnp.arange(N_TOK, dtype=jnp.int32)
    )
    g = ridx // E_LOCAL
    src = g * C + (inv - gstart[g])
    out = blocks.reshape(N_DEV * C, H)[src]
    return out.astype(jnp.float32)


# device time: 227517 ns/iter; 1.3476x vs baseline; 1.3476x over previous
import jax
import jax.numpy as jnp
from jax import lax
from jax.experimental import pallas as pl
from jax.experimental.pallas import tpu as pltpu

N_DEV = 4
N_TOK = 4096
D_MODEL = 1024
H = 2048
HB = H // 2
E_LOCAL = 4
CHUNK = N_TOK // N_DEV
N_STEPS = 2 * (N_DEV - 1)


def kernel(x, router_W, route_idx, expert_W):
    del router_W
    xb = x.astype(jnp.bfloat16)
    wb = expert_W.astype(jnp.bfloat16).reshape(E_LOCAL * D_MODEL, H)

    def body(x_ref, idx_ref, w_ref, out_ref, comm_cw, comm_ccw, xm_ref,
             acc_cw, acc_ccw, send_cw, recv_cw, send_ccw, recv_ccw, copy_sems):
        my = lax.axis_index("i")
        left = lax.rem(my + N_DEV - 1, N_DEV)
        right = lax.rem(my + 1, N_DEV)

        barrier_sem = pltpu.get_barrier_semaphore()
        for nbr in (left, right):
            pl.semaphore_signal(
                barrier_sem, inc=1,
                device_id=(nbr,), device_id_type=pl.DeviceIdType.MESH,
            )
        pl.semaphore_wait(barrier_sem, 2)

        def build_xcat(c):
            xc = x_ref[pl.ds(c * CHUNK, CHUNK), :]
            ic = idx_ref[pl.ds(c * CHUNK, CHUNK), :]
            for le in range(E_LOCAL):
                xm_ref[:, pl.ds(le * D_MODEL, D_MODEL)] = jnp.where(
                    ic == my * E_LOCAL + le, xc, jnp.zeros_like(xc)
                )

        def matmul_half(dst, half):
            dst[...] = jnp.dot(
                xm_ref[...],
                w_ref[:, pl.ds(half * HB, HB)],
                preferred_element_type=jnp.float32,
            ).astype(jnp.bfloat16)

        def partial_pair(c_cw, c_ccw, shared, dst_cw, dst_ccw):
            build_xcat(c_cw)
            matmul_half(dst_cw, 0)
            if not shared:
                build_xcat(c_ccw)
            matmul_half(dst_ccw, 1)

        def store_half(c, src, half, sem_idx):
            cp = pltpu.make_async_copy(
                src,
                out_ref.at[pl.ds(c * CHUNK, CHUNK), pl.ds(half * HB, HB)],
                copy_sems.at[sem_idx],
            )
            cp.start()
            return cp

        def make_rdmas(s):
            send_slot = s % 3
            recv_slot = (s + 1) % 3
            rdma_cw = pltpu.make_async_remote_copy(
                src_ref=comm_cw.at[send_slot],
                dst_ref=comm_cw.at[recv_slot],
                send_sem=send_cw.at[s],
                recv_sem=recv_cw.at[s],
                device_id=(right,),
                device_id_type=pl.DeviceIdType.MESH,
            )
            rdma_ccw = pltpu.make_async_remote_copy(
                src_ref=comm_ccw.at[send_slot],
                dst_ref=comm_ccw.at[recv_slot],
                send_sem=send_ccw.at[s],
                recv_sem=recv_ccw.at[s],
                device_id=(left,),
                device_id_type=pl.DeviceIdType.MESH,
            )
            return rdma_cw, rdma_ccw

        build_xcat(my)
        matmul_half(comm_cw.at[0], 0)
        cur_cw, cur_ccw = make_rdmas(0)
        cur_cw.start()
        matmul_half(comm_ccw.at[0], 1)
        cur_ccw.start()

        pending = []
        for s in range(N_STEPS):
            recv_slot = (s + 1) % 3
            if s < N_DEV - 1:
                cr_cw = lax.rem(my - 1 - s + 2 * N_DEV, N_DEV)
                cr_ccw = lax.rem(my + 1 + s, N_DEV)
                partial_pair(cr_cw, cr_ccw, s % 2 == 1, acc_cw, acc_ccw)
                cur_cw.wait()
                cur_ccw.wait()
                comm_cw[recv_slot, :, :] += acc_cw[...]
                comm_ccw[recv_slot, :, :] += acc_ccw[...]
            else:
                t = s - (N_DEV - 1)
                cr_cw = lax.rem(my - t + N_DEV, N_DEV)
                cr_ccw = lax.rem(my + t, N_DEV)
                cur_cw.wait()
                cur_ccw.wait()
            if s + 1 < N_STEPS:
                cur_cw, cur_ccw = make_rdmas(s + 1)
                cur_cw.start()
                cur_ccw.start()
            if s >= N_DEV - 2:
                pending.append(
                    store_half(cr_cw, comm_cw.at[recv_slot], 0, len(pending)))
                pending.append(
                    store_half(cr_ccw, comm_ccw.at[recv_slot], 1, len(pending)))
        for cp in pending:
            cp.wait()

    out = pl.pallas_call(
        body,
        out_shape=jax.ShapeDtypeStruct((N_TOK, H), jnp.bfloat16),
        in_specs=[
            pl.BlockSpec(memory_space=pltpu.VMEM),
            pl.BlockSpec(memory_space=pltpu.VMEM),
            pl.BlockSpec(memory_space=pltpu.VMEM),
        ],
        out_specs=pl.BlockSpec(memory_space=pl.ANY),
        scratch_shapes=[
            pltpu.VMEM((3, CHUNK, HB), jnp.bfloat16),
            pltpu.VMEM((3, CHUNK, HB), jnp.bfloat16),
            pltpu.VMEM((CHUNK, E_LOCAL * D_MODEL), jnp.bfloat16),
            pltpu.VMEM((CHUNK, HB), jnp.bfloat16),
            pltpu.VMEM((CHUNK, HB), jnp.bfloat16),
            pltpu.SemaphoreType.DMA((N_STEPS,)),
            pltpu.SemaphoreType.DMA((N_STEPS,)),
            pltpu.SemaphoreType.DMA((N_STEPS,)),
            pltpu.SemaphoreType.DMA((N_STEPS,)),
            pltpu.SemaphoreType.DMA((8,)),
        ],
        compiler_params=pltpu.CompilerParams(
            collective_id=0, vmem_limit_bytes=100 * 1024 * 1024
        ),
    )(xb, route_idx, wb)
    return out.astype(jnp.float32)


# device time: 222760 ns/iter; 1.3764x vs baseline; 1.0214x over previous
import jax
import jax.numpy as jnp
from jax import lax
from jax.experimental import pallas as pl
from jax.experimental.pallas import tpu as pltpu

N_DEV = 4
N_TOK = 4096
D_MODEL = 1024
H = 2048
HB = H // 2
E_LOCAL = 4
CHUNK = N_TOK // N_DEV
N_STEPS = 2 * (N_DEV - 1)


def kernel(x, router_W, route_idx, expert_W):
    del router_W
    xb = x.astype(jnp.bfloat16)
    wb = expert_W.astype(jnp.bfloat16).reshape(E_LOCAL * D_MODEL, H)

    def body(x_ref, idx_ref, w_ref, out_ref, comm_cw, comm_ccw, xm_ref,
             acc_cw, acc_ccw, send_cw, recv_cw, send_ccw, recv_ccw, copy_sems):
        my = lax.axis_index("i")
        left = lax.rem(my + N_DEV - 1, N_DEV)
        right = lax.rem(my + 1, N_DEV)

        barrier_sem = pltpu.get_barrier_semaphore()
        for nbr in (left, right):
            pl.semaphore_signal(
                barrier_sem, inc=1,
                device_id=(nbr,), device_id_type=pl.DeviceIdType.MESH,
            )
        pl.semaphore_wait(barrier_sem, 2)

        def build_xcat(c):
            xc = x_ref[pl.ds(c * CHUNK, CHUNK), :]
            ic = idx_ref[pl.ds(c * CHUNK, CHUNK), :]
            for le in range(E_LOCAL):
                xm_ref[:, pl.ds(le * D_MODEL, D_MODEL)] = jnp.where(
                    ic == my * E_LOCAL + le, xc, jnp.zeros_like(xc)
                )

        def matmul_half(dst, half):
            dst[...] = jnp.dot(
                xm_ref[...],
                w_ref[:, pl.ds(half * HB, HB)],
                preferred_element_type=jnp.float32,
            ).astype(jnp.bfloat16)

        def partial_pair(c_cw, c_ccw, shared, dst_cw, dst_ccw):
            build_xcat(c_cw)
            matmul_half(dst_cw, 0)
            if not shared:
                build_xcat(c_ccw)
            matmul_half(dst_ccw, 1)

        def store_half(c, src, half, sem_idx):
            cp = pltpu.make_async_copy(
                src,
                out_ref.at[pl.ds(c * CHUNK, CHUNK), pl.ds(half * HB, HB)],
                copy_sems.at[sem_idx],
            )
            cp.start()
            return cp

        def make_rdmas(s):
            send_slot = s % 3
            recv_slot = (s + 1) % 3
            rdma_cw = pltpu.make_async_remote_copy(
                src_ref=comm_cw.at[send_slot],
                dst_ref=comm_cw.at[recv_slot],
                send_sem=send_cw.at[s],
                recv_sem=recv_cw.at[s],
                device_id=(right,),
                device_id_type=pl.DeviceIdType.MESH,
            )
            rdma_ccw = pltpu.make_async_remote_copy(
                src_ref=comm_ccw.at[send_slot],
                dst_ref=comm_ccw.at[recv_slot],
                send_sem=send_ccw.at[s],
                recv_sem=recv_ccw.at[s],
                device_id=(left,),
                device_id_type=pl.DeviceIdType.MESH,
            )
            return rdma_cw, rdma_ccw

        build_xcat(my)
        matmul_half(comm_cw.at[0], 0)
        cur_cw, cur_ccw = make_rdmas(0)
        cur_cw.start()
        matmul_half(comm_ccw.at[0], 1)
        cur_ccw.start()

        pending = []
        for s in range(N_STEPS):
            recv_slot = (s + 1) % 3
            if s < N_DEV - 1:
                cr_cw = lax.rem(my - 1 - s + 2 * N_DEV, N_DEV)
                cr_ccw = lax.rem(my + 1 + s, N_DEV)
                partial_pair(cr_cw, cr_ccw, s % 2 == 1, acc_cw, acc_ccw)
                cur_cw.wait()
                cur_ccw.wait()
                comm_cw[recv_slot, :, :] += acc_cw[...]
                comm_ccw[recv_slot, :, :] += acc_ccw[...]
            else:
                t = s - (N_DEV - 1)
                cr_cw = lax.rem(my - t + N_DEV, N_DEV)
                cr_ccw = lax.rem(my + t, N_DEV)
                cur_cw.wait()
                cur_ccw.wait()
            if s + 1 < N_STEPS:
                cur_cw, cur_ccw = make_rdmas(s + 1)
                cur_cw.start()
                cur_ccw.start()
            if s >= N_DEV - 2:
                pending.append(
                    store_half(cr_cw, comm_cw.at[recv_slot], 0, len(pending)))
                pending.append(
                    store_half(cr_ccw, comm_ccw.at[recv_slot], 1, len(pending)))
        for cp in pending:
            cp.wait()

    out = pl.pallas_call(
        body,
        out_shape=jax.ShapeDtypeStruct((N_TOK, H), jnp.bfloat16),
        in_specs=[
            pl.BlockSpec(memory_space=pltpu.VMEM),
            pl.BlockSpec(memory_space=pltpu.VMEM),
            pl.BlockSpec(memory_space=pltpu.VMEM),
        ],
        out_specs=pl.BlockSpec(memory_space=pl.ANY),
        scratch_shapes=[
            pltpu.VMEM((3, CHUNK, HB), jnp.bfloat16),
            pltpu.VMEM((3, CHUNK, HB), jnp.bfloat16),
            pltpu.VMEM((CHUNK, E_LOCAL * D_MODEL), jnp.bfloat16),
            pltpu.VMEM((CHUNK, HB), jnp.bfloat16),
            pltpu.VMEM((CHUNK, HB), jnp.bfloat16),
            pltpu.SemaphoreType.DMA((N_STEPS,)),
            pltpu.SemaphoreType.DMA((N_STEPS,)),
            pltpu.SemaphoreType.DMA((N_STEPS,)),
            pltpu.SemaphoreType.DMA((N_STEPS,)),
            pltpu.SemaphoreType.DMA((8,)),
        ],
        compiler_params=pltpu.CompilerParams(
            collective_id=0, vmem_limit_bytes=100 * 1024 * 1024
        ),
    )(xb, route_idx, wb)
    return out
